# Initial kernel scaffold; baseline (speedup 1.0000x reference)
import jax
import jax.numpy as jnp
from jax.experimental import pallas as pl


def kernel(y_recv, x_flat, gate, local_eid, W1, b1, W2, b2):
    raise NotImplementedError("write your pallas kernel here")


# speedup vs baseline: 3.1777x; 3.1777x over previous
import functools
import jax
import jax.numpy as jnp
from jax.experimental import pallas as pl
from jax.experimental.pallas import tpu as pltpu

TB = 32
FB = 512
PAD = 256


def _ffn_body(off_ref, x_ref, g_ref, w1_ref, b1_ref, w2_ref, b2_ref, out_ref):
    e = pl.program_id(0)
    f = pl.program_id(1)

    @pl.when((e == 0) & (f == 0))
    def _init():
        out_ref[...] = jnp.zeros_like(out_ref)

    start = off_ref[e]
    end = off_ref[e + 1]
    base = (start // 8) * 8
    num_tiles = (end - base + TB - 1) // TB

    w1 = w1_ref[0]
    b1 = b1_ref[0]
    w2 = w2_ref[0]
    b2 = b2_ref[0]

    def body(k, carry):
        p = pl.multiple_of(base + k * TB, 8)
        x = x_ref[pl.ds(p, TB), :]
        g = g_ref[pl.ds(p, TB), :]
        h = jnp.maximum(jnp.dot(x, w1, preferred_element_type=jnp.float32) + b1, 0.0)
        y = jnp.dot(h, w2, preferred_element_type=jnp.float32)
        contrib = y * g
        contrib = jnp.where(f == 0, contrib + b2 * g, contrib)
        row = p + jax.lax.broadcasted_iota(jnp.int32, (TB, 1), 0)
        contrib = jnp.where((row >= start) & (row < end), contrib, 0.0)
        out_ref[pl.ds(p, TB), :] += contrib
        return carry

    jax.lax.fori_loop(0, num_tiles, body, 0)


def _ffn(off, x_s, g_s, W1, b1, W2, b2):
    E, D, F = W1.shape
    T_pad = x_s.shape[0]
    k_ff = F // FB
    grid_spec = pltpu.PrefetchScalarGridSpec(
        num_scalar_prefetch=1,
        grid=(E, k_ff),
        in_specs=[
            pl.BlockSpec((T_pad, D), lambda e, f, off: (0, 0)),
            pl.BlockSpec((T_pad, 1), lambda e, f, off: (0, 0)),
            pl.BlockSpec((1, D, FB), lambda e, f, off: (e, 0, f)),
            pl.BlockSpec((1, 1, FB), lambda e, f, off: (e, 0, f)),
            pl.BlockSpec((1, FB, D), lambda e, f, off: (e, f, 0)),
            pl.BlockSpec((1, 1, D), lambda e, f, off: (e, 0, 0)),
        ],
        out_specs=pl.BlockSpec((T_pad, D), lambda e, f, off: (0, 0)),
    )
    return pl.pallas_call(
        _ffn_body,
        grid_spec=grid_spec,
        out_shape=jax.ShapeDtypeStruct((T_pad, D), jnp.float32),
        compiler_params=pltpu.CompilerParams(
            dimension_semantics=("arbitrary", "arbitrary"),
        ),
    )(off, x_s, g_s, W1, b1.reshape(E, 1, F), W2, b2.reshape(E, 1, D))


def kernel(y_recv, x_flat, gate, local_eid, W1, b1, W2, b2):
    T, D = x_flat.shape
    E = W1.shape[0]
    T_pad = T + PAD

    eid = local_eid.astype(jnp.int32)
    perm = jnp.argsort(eid, stable=True).astype(jnp.int32)
    counts = jnp.zeros((E,), jnp.int32).at[eid].add(1)
    off = jnp.concatenate([jnp.zeros((1,), jnp.int32), jnp.cumsum(counts)])
    off = off.astype(jnp.int32)

    perm_p = jnp.concatenate([perm, jnp.zeros((T_pad - T,), jnp.int32)])
    x_s = x_flat[perm_p]
    g_s = gate[perm_p][:, None]

    y_s = _ffn(off, x_s, g_s, W1, b1, W2, b2)
    return y_recv.at[perm].set(y_s[:T])


# speedup vs baseline: 3.9294x; 1.2366x over previous
import functools
import jax
import jax.numpy as jnp
from jax.experimental import pallas as pl
from jax.experimental.pallas import tpu as pltpu

TB = 32
FB = 1024
PAD = 256


def _ffn_body(off_ref, x_ref, g_ref, w1_ref, b1_ref, w2_ref, b2_ref, out_ref):
    e = pl.program_id(0)
    f = pl.program_id(1)

    @pl.when((e == 0) & (f == 0))
    def _init():
        out_ref[...] = jnp.zeros_like(out_ref)

    start = off_ref[e]
    end = off_ref[e + 1]
    base = (start // 8) * 8
    num_tiles = (end - base + TB - 1) // TB

    w1 = w1_ref[0]
    b1 = b1_ref[0]
    w2 = w2_ref[0]
    b2 = b2_ref[0]

    def body(k, carry):
        p = pl.multiple_of(base + k * TB, 8)
        x = x_ref[pl.ds(p, TB), :]
        g = g_ref[pl.ds(p, TB), :]
        h = jnp.maximum(jnp.dot(x, w1, preferred_element_type=jnp.float32) + b1, 0.0)
        y = jnp.dot(h, w2, preferred_element_type=jnp.float32)
        contrib = y * g
        contrib = jnp.where(f == 0, contrib + b2 * g, contrib)
        row = p + jax.lax.broadcasted_iota(jnp.int32, (TB, 1), 0)
        contrib = jnp.where((row >= start) & (row < end), contrib, 0.0)
        out_ref[pl.ds(p, TB), :] += contrib
        return carry

    jax.lax.fori_loop(0, num_tiles, body, 0)


def _ffn(off, x_s, g_s, W1, b1, W2, b2):
    E, D, F = W1.shape
    T_pad = x_s.shape[0]
    k_ff = F // FB
    grid_spec = pltpu.PrefetchScalarGridSpec(
        num_scalar_prefetch=1,
        grid=(E, k_ff),
        in_specs=[
            pl.BlockSpec((T_pad, D), lambda e, f, off: (0, 0)),
            pl.BlockSpec((T_pad, 1), lambda e, f, off: (0, 0)),
            pl.BlockSpec((1, D, FB), lambda e, f, off: (e, 0, f)),
            pl.BlockSpec((1, 1, FB), lambda e, f, off: (e, 0, f)),
            pl.BlockSpec((1, FB, D), lambda e, f, off: (e, f, 0)),
            pl.BlockSpec((1, 1, D), lambda e, f, off: (e, 0, 0)),
        ],
        out_specs=pl.BlockSpec((T_pad, D), lambda e, f, off: (0, 0)),
    )
    return pl.pallas_call(
        _ffn_body,
        grid_spec=grid_spec,
        out_shape=jax.ShapeDtypeStruct((T_pad, D), jnp.float32),
        compiler_params=pltpu.CompilerParams(
            dimension_semantics=("arbitrary", "arbitrary"),
        ),
    )(off, x_s, g_s, W1, b1.reshape(E, 1, F), W2, b2.reshape(E, 1, D))


def kernel(y_recv, x_flat, gate, local_eid, W1, b1, W2, b2):
    T, D = x_flat.shape
    E = W1.shape[0]
    T_pad = T + PAD

    eid = local_eid.astype(jnp.int32)
    perm = jnp.argsort(eid, stable=True).astype(jnp.int32)
    counts = jnp.zeros((E,), jnp.int32).at[eid].add(1)
    off = jnp.concatenate([jnp.zeros((1,), jnp.int32), jnp.cumsum(counts)])
    off = off.astype(jnp.int32)

    perm_p = jnp.concatenate([perm, jnp.zeros((T_pad - T,), jnp.int32)])
    x_s = x_flat[perm_p]
    g_s = gate[perm_p][:, None]

    y_s = _ffn(off, x_s, g_s, W1, b1, W2, b2)
    return y_recv.at[perm].set(y_s[:T])


# speedup vs baseline: 4.1517x; 1.0565x over previous
import functools
import jax
import jax.numpy as jnp
from jax.experimental import pallas as pl
from jax.experimental.pallas import tpu as pltpu

TB = 32
FB = 2048
PAD = 64


def _ffn_body(off_ref, x_ref, g_ref, w1_ref, b1_ref, w2_ref, b2_ref, out_ref):
    e = pl.program_id(0)
    f = pl.program_id(1)

    @pl.when((e == 0) & (f == 0))
    def _init():
        out_ref[...] = jnp.zeros_like(out_ref)

    start = off_ref[e]
    end = off_ref[e + 1]
    base = (start // 8) * 8
    num_tiles = (end - base + TB - 1) // TB

    w1 = w1_ref[0]
    b1 = b1_ref[0]
    w2 = w2_ref[0]
    b2 = b2_ref[0]

    def body(k, carry):
        p = pl.multiple_of(base + k * TB, 8)
        x = x_ref[pl.ds(p, TB), :]
        g = g_ref[pl.ds(p, TB), :]
        h = jnp.maximum(jnp.dot(x, w1, preferred_element_type=jnp.float32) + b1, 0.0)
        y = jnp.dot(h, w2, preferred_element_type=jnp.float32)
        contrib = y * g
        contrib = jnp.where(f == 0, contrib + b2 * g, contrib)
        row = p + jax.lax.broadcasted_iota(jnp.int32, (TB, 1), 0)
        contrib = jnp.where((row >= start) & (row < end), contrib, 0.0)
        out_ref[pl.ds(p, TB), :] += contrib
        return carry

    jax.lax.fori_loop(0, num_tiles, body, 0)


def _ffn(off, x_s, g_s, W1, b1, W2, b2):
    E, D, F = W1.shape
    T_pad = x_s.shape[0]
    k_ff = F // FB
    grid_spec = pltpu.PrefetchScalarGridSpec(
        num_scalar_prefetch=1,
        grid=(E, k_ff),
        in_specs=[
            pl.BlockSpec((T_pad, D), lambda e, f, off: (0, 0)),
            pl.BlockSpec((T_pad, 1), lambda e, f, off: (0, 0)),
            pl.BlockSpec((1, D, FB), lambda e, f, off: (e, 0, f)),
            pl.BlockSpec((1, 1, FB), lambda e, f, off: (e, 0, f)),
            pl.BlockSpec((1, FB, D), lambda e, f, off: (e, f, 0)),
            pl.BlockSpec((1, 1, D), lambda e, f, off: (e, 0, 0)),
        ],
        out_specs=pl.BlockSpec((T_pad, D), lambda e, f, off: (0, 0)),
    )
    return pl.pallas_call(
        _ffn_body,
        grid_spec=grid_spec,
        out_shape=jax.ShapeDtypeStruct((T_pad, D), jnp.float32),
        compiler_params=pltpu.CompilerParams(
            dimension_semantics=("arbitrary", "arbitrary"),
        ),
    )(off, x_s, g_s, W1, b1.reshape(E, 1, F), W2, b2.reshape(E, 1, D))


def kernel(y_recv, x_flat, gate, local_eid, W1, b1, W2, b2):
    T, D = x_flat.shape
    E = W1.shape[0]
    T_pad = T + PAD

    eid = local_eid.astype(jnp.int32)
    perm = jnp.argsort(eid, stable=True).astype(jnp.int32)
    counts = jnp.zeros((E,), jnp.int32).at[eid].add(1)
    off = jnp.concatenate([jnp.zeros((1,), jnp.int32), jnp.cumsum(counts)])
    off = off.astype(jnp.int32)

    perm_p = jnp.concatenate([perm, jnp.zeros((T_pad - T,), jnp.int32)])
    x_s = x_flat[perm_p]
    g_s = gate[perm_p][:, None]

    y_s = _ffn(off, x_s, g_s, W1, b1, W2, b2)
    return y_recv.at[perm].set(y_s[:T])


# speedup vs baseline: 4.2592x; 1.0259x over previous
import functools
import jax
import jax.numpy as jnp
from jax import lax
from jax.experimental import pallas as pl
from jax.experimental.pallas import tpu as pltpu
from jax.experimental.pallas import tpu_sc as plsc

TB = 32
FB = 2048
PAD = 64


def _ffn_body(off_ref, x_ref, g_ref, w1_ref, b1_ref, w2_ref, b2_ref, out_ref):
    e = pl.program_id(0)
    f = pl.program_id(1)

    @pl.when((e == 0) & (f == 0))
    def _init():
        out_ref[...] = jnp.zeros_like(out_ref)

    start = off_ref[e]
    end = off_ref[e + 1]
    base = (start // 8) * 8
    num_tiles = (end - base + TB - 1) // TB

    w1 = w1_ref[0]
    b1 = b1_ref[0]
    w2 = w2_ref[0]
    b2 = b2_ref[0]

    def body(k, carry):
        p = pl.multiple_of(base + k * TB, 8)
        x = x_ref[pl.ds(p, TB), :]
        g = g_ref[pl.ds(p, TB), :]
        h = jnp.maximum(jnp.dot(x, w1, preferred_element_type=jnp.float32) + b1, 0.0)
        y = jnp.dot(h, w2, preferred_element_type=jnp.float32)
        contrib = y * g
        contrib = jnp.where(f == 0, contrib + b2 * g, contrib)
        row = p + jax.lax.broadcasted_iota(jnp.int32, (TB, 1), 0)
        contrib = jnp.where((row >= start) & (row < end), contrib, 0.0)
        out_ref[pl.ds(p, TB), :] += contrib
        return carry

    jax.lax.fori_loop(0, num_tiles, body, 0)


def _ffn(off, x_s, g_s, W1, b1, W2, b2):
    E, D, F = W1.shape
    T_pad = x_s.shape[0]
    k_ff = F // FB
    grid_spec = pltpu.PrefetchScalarGridSpec(
        num_scalar_prefetch=1,
        grid=(E, k_ff),
        in_specs=[
            pl.BlockSpec((T_pad, D), lambda e, f, off: (0, 0)),
            pl.BlockSpec((T_pad, 1), lambda e, f, off: (0, 0)),
            pl.BlockSpec((1, D, FB), lambda e, f, off: (e, 0, f)),
            pl.BlockSpec((1, 1, FB), lambda e, f, off: (e, 0, f)),
            pl.BlockSpec((1, FB, D), lambda e, f, off: (e, f, 0)),
            pl.BlockSpec((1, 1, D), lambda e, f, off: (e, 0, 0)),
        ],
        out_specs=pl.BlockSpec((T_pad, D), lambda e, f, off: (0, 0)),
    )
    return pl.pallas_call(
        _ffn_body,
        grid_spec=grid_spec,
        out_shape=jax.ShapeDtypeStruct((T_pad, D), jnp.float32),
        compiler_params=pltpu.CompilerParams(
            dimension_semantics=("arbitrary", "arbitrary"),
        ),
    )(off, x_s, g_s, W1, b1.reshape(E, 1, F), W2, b2.reshape(E, 1, D))


def _sc_gather(x_flat, gate, perm, T_pad):
    T, D = x_flat.shape
    info = plsc.get_sparse_core_info()
    NC, NS, L = info.num_cores, info.num_subcores, info.num_lanes
    NW = NC * NS
    bpw = T // NW
    mesh = plsc.VectorSubcoreMesh(core_axis_name="c", subcore_axis_name="s")

    @functools.partial(
        pl.kernel,
        mesh=mesh,
        out_type=[
            jax.ShapeDtypeStruct((T_pad, D), jnp.float32),
            jax.ShapeDtypeStruct((T_pad,), jnp.float32),
        ],
        scratch_types=[
            pltpu.VMEM((bpw,), jnp.int32),
            pltpu.VMEM((bpw, D), jnp.float32),
            pltpu.VMEM((bpw,), jnp.float32),
            pltpu.SemaphoreType.DMA,
            pltpu.SemaphoreType.DMA,
        ],
    )
    def k(x_hbm, gate_hbm, perm_hbm, xs_hbm, gs_hbm, idx_v, rows_v, gg_v, sem, sem2):
        wid = lax.axis_index("s") * NC + lax.axis_index("c")
        base = wid * bpw
        pltpu.sync_copy(perm_hbm.at[pl.ds(base, bpw)], idx_v)
        cp = pltpu.async_copy(x_hbm.at[idx_v], rows_v, sem)
        pltpu.async_copy(gate_hbm.at[idx_v], gg_v, sem2).wait()
        cp.wait()
        pltpu.sync_copy(rows_v, xs_hbm.at[pl.ds(base, bpw)])
        pltpu.sync_copy(gg_v, gs_hbm.at[pl.ds(base, bpw)])

    return k(x_flat, gate, perm)


def _sc_scatter(y_s, perm, T, D):
    info = plsc.get_sparse_core_info()
    NC, NS = info.num_cores, info.num_subcores
    NW = NC * NS
    bpw = T // NW
    mesh = plsc.VectorSubcoreMesh(core_axis_name="c", subcore_axis_name="s")

    @functools.partial(
        pl.kernel,
        mesh=mesh,
        out_type=jax.ShapeDtypeStruct((T, D), jnp.float32),
        scratch_types=[
            pltpu.VMEM((bpw,), jnp.int32),
            pltpu.VMEM((bpw, D), jnp.float32),
            pltpu.SemaphoreType.DMA,
        ],
    )
    def k(ys_hbm, perm_hbm, out_hbm, idx_v, rows_v, sem):
        wid = lax.axis_index("s") * NC + lax.axis_index("c")
        base = wid * bpw
        pltpu.sync_copy(perm_hbm.at[pl.ds(base, bpw)], idx_v)
        pltpu.sync_copy(ys_hbm.at[pl.ds(base, bpw)], rows_v)
        pltpu.async_copy(rows_v, out_hbm.at[idx_v], sem).wait()

    return k(y_s, perm)


def kernel(y_recv, x_flat, gate, local_eid, W1, b1, W2, b2):
    T, D = x_flat.shape
    E = W1.shape[0]
    T_pad = T + PAD

    eid = local_eid.astype(jnp.int32)
    perm = jnp.argsort(eid, stable=True).astype(jnp.int32)
    counts = jnp.zeros((E,), jnp.int32).at[eid].add(1)
    off = jnp.concatenate([jnp.zeros((1,), jnp.int32), jnp.cumsum(counts)])
    off = off.astype(jnp.int32)

    x_s, g_s = _sc_gather(x_flat, gate, perm, T_pad)
    y_s = _ffn(off, x_s, g_s.reshape(T_pad, 1), W1, b1, W2, b2)
    return _sc_scatter(y_s, perm, T, D)
